# initial kernel scaffold (unmeasured)
import jax
import jax.numpy as jnp
from jax import lax
from jax.experimental import pallas as pl
from jax.experimental.pallas import tpu as pltpu


def kernel(ids, E):
    (T,) = ids.shape
    Vp, D = E.shape
    Th = T // 2

    def body(ids_ref, e_ref, out_ref,
             xsend, xrecv, ysend, yrecv,
             sem_xs, sem_xr, sem_ys, sem_yr):
        my_x = lax.axis_index("x")
        my_y = lax.axis_index("y")
        nx = 1 - my_x
        ny = 1 - my_y

        barrier_sem = pltpu.get_barrier_semaphore()
        pl.semaphore_signal(barrier_sem, inc=1, device_id=(nx, my_y),
                            device_id_type=pl.DeviceIdType.MESH)
        pl.semaphore_signal(barrier_sem, inc=1, device_id=(my_x, ny),
                            device_id_type=pl.DeviceIdType.MESH)
        pl.semaphore_wait(barrier_sem, 2)

        base = my_y * Th
        voff = my_x * Vp

        def gather_one(t, _):
            idx = ids_ref[base + t]
            rel = idx - voff
            valid = jnp.logical_and(rel >= 0, rel < Vp)
            safe = jnp.where(valid, rel, 0)
            row = pl.load(e_ref, (pl.ds(safe, 1), slice(None)))
            row = jnp.where(valid, row, 0.0).astype(jnp.bfloat16)
            pl.store(xsend, (pl.ds(t, 1), slice(None)), row)
            return 0

        lax.fori_loop(0, Th, gather_one, 0)

        rdma_x = pltpu.make_async_remote_copy(
            src_ref=xsend, dst_ref=xrecv,
            send_sem=sem_xs, recv_sem=sem_xr,
            device_id=(nx, my_y), device_id_type=pl.DeviceIdType.MESH,
        )
        rdma_x.start()
        rdma_x.wait()

        ysend[...] = xsend[...] + xrecv[...]
        out_ref[pl.ds(base, Th), :] = ysend[...].astype(jnp.float32)

        rdma_y = pltpu.make_async_remote_copy(
            src_ref=ysend, dst_ref=yrecv,
            send_sem=sem_ys, recv_sem=sem_yr,
            device_id=(my_x, ny), device_id_type=pl.DeviceIdType.MESH,
        )
        rdma_y.start()
        rdma_y.wait()

        out_ref[pl.ds(ny * Th, Th), :] = yrecv[...].astype(jnp.float32)

    return pl.pallas_call(
        body,
        out_shape=jax.ShapeDtypeStruct((T, D), jnp.float32),
        in_specs=[
            pl.BlockSpec(memory_space=pltpu.SMEM),
            pl.BlockSpec(memory_space=pltpu.VMEM),
        ],
        out_specs=pl.BlockSpec(memory_space=pltpu.VMEM),
        scratch_shapes=[
            pltpu.VMEM((Th, D), jnp.bfloat16),
            pltpu.VMEM((Th, D), jnp.bfloat16),
            pltpu.VMEM((Th, D), jnp.bfloat16),
            pltpu.VMEM((Th, D), jnp.bfloat16),
            pltpu.SemaphoreType.DMA,
            pltpu.SemaphoreType.DMA,
            pltpu.SemaphoreType.DMA,
            pltpu.SemaphoreType.DMA,
        ],
        compiler_params=pltpu.CompilerParams(collective_id=0),
    )(ids, E)


# baseline (device time: 48685 ns/iter reference)
import jax
import jax.numpy as jnp
from jax import lax
from jax.experimental import pallas as pl
from jax.experimental.pallas import tpu as pltpu


def kernel(ids, E):
    (T,) = ids.shape
    Vp, D = E.shape
    Th = T // 2

    ids2d = ids.reshape(T, 1)

    def body(ids_ref, idv_ref, e_ref, out_ref,
             stage, xsend, xrecv, ysend, yrecv,
             sem_g, sem_xs, sem_xr, sem_ys, sem_yr):
        my_x = lax.axis_index("x")
        my_y = lax.axis_index("y")
        nx = 1 - my_x
        ny = 1 - my_y

        barrier_sem = pltpu.get_barrier_semaphore()
        pl.semaphore_signal(barrier_sem, inc=1, device_id=(nx, my_y),
                            device_id_type=pl.DeviceIdType.MESH)
        pl.semaphore_signal(barrier_sem, inc=1, device_id=(my_x, ny),
                            device_id_type=pl.DeviceIdType.MESH)
        pl.semaphore_wait(barrier_sem, 2)

        base = my_y * Th
        voff = my_x * Vp

        def issue_one(t, _):
            idx = ids_ref[base + t]
            rel = idx - voff
            safe = jnp.minimum(jnp.maximum(rel, 0), Vp - 1)
            pltpu.make_async_copy(
                e_ref.at[pl.ds(safe, 1), :],
                stage.at[pl.ds(t, 1), :],
                sem_g,
            ).start()
            return 0

        lax.fori_loop(0, Th, issue_one, 0)

        def wait_one(t, _):
            pltpu.make_async_copy(
                e_ref.at[pl.ds(0, 1), :],
                stage.at[pl.ds(0, 1), :],
                sem_g,
            ).wait()
            return 0

        lax.fori_loop(0, Th, wait_one, 0)

        idv = idv_ref[pl.ds(base, Th), :]
        mask = jnp.logical_and(idv >= voff, idv < voff + Vp)
        xsend[...] = jnp.where(mask, stage[...], 0.0).astype(jnp.bfloat16)

        rdma_x = pltpu.make_async_remote_copy(
            src_ref=xsend, dst_ref=xrecv,
            send_sem=sem_xs, recv_sem=sem_xr,
            device_id=(nx, my_y), device_id_type=pl.DeviceIdType.MESH,
        )
        rdma_x.start()
        rdma_x.wait()

        ysend[...] = xsend[...] + xrecv[...]
        out_ref[pl.ds(base, Th), :] = ysend[...].astype(jnp.float32)

        rdma_y = pltpu.make_async_remote_copy(
            src_ref=ysend, dst_ref=yrecv,
            send_sem=sem_ys, recv_sem=sem_yr,
            device_id=(my_x, ny), device_id_type=pl.DeviceIdType.MESH,
        )
        rdma_y.start()
        rdma_y.wait()

        out_ref[pl.ds(ny * Th, Th), :] = yrecv[...].astype(jnp.float32)

    return pl.pallas_call(
        body,
        out_shape=jax.ShapeDtypeStruct((T, D), jnp.float32),
        in_specs=[
            pl.BlockSpec(memory_space=pltpu.SMEM),
            pl.BlockSpec(memory_space=pltpu.VMEM),
            pl.BlockSpec(memory_space=pl.ANY),
        ],
        out_specs=pl.BlockSpec(memory_space=pltpu.VMEM),
        scratch_shapes=[
            pltpu.VMEM((Th, D), jnp.float32),
            pltpu.VMEM((Th, D), jnp.bfloat16),
            pltpu.VMEM((Th, D), jnp.bfloat16),
            pltpu.VMEM((Th, D), jnp.bfloat16),
            pltpu.VMEM((Th, D), jnp.bfloat16),
            pltpu.SemaphoreType.DMA,
            pltpu.SemaphoreType.DMA,
            pltpu.SemaphoreType.DMA,
            pltpu.SemaphoreType.DMA,
            pltpu.SemaphoreType.DMA,
        ],
        compiler_params=pltpu.CompilerParams(collective_id=0),
    )(ids, ids2d, E)


# device time: 31223 ns/iter; 1.5593x vs baseline; 1.5593x over previous
import jax
import jax.numpy as jnp
from jax import lax
from jax.experimental import pallas as pl
from jax.experimental.pallas import tpu as pltpu

C = 8


def kernel(ids, E):
    (T,) = ids.shape
    Vp, D = E.shape
    Th = T // 2
    S = Th // C

    ids2d = ids.reshape(T, 1)

    def body(ids_ref, idv_ref, e_ref, out_ref,
             stage, xsend, xrecv, ysend, yrecv, obuf,
             sem_g, sem_o, sem_xs, sem_xr, sem_ys, sem_yr):
        my_x = lax.axis_index("x")
        my_y = lax.axis_index("y")
        nx = 1 - my_x
        ny = 1 - my_y

        base = my_y * Th
        voff = my_x * Vp

        def issue_chunk(c):
            def issue_one(j, _):
                t = c * S + j
                idx = ids_ref[base + t]
                rel = idx - voff
                safe = jnp.minimum(jnp.maximum(rel, 0), Vp - 1)
                pltpu.make_async_copy(
                    e_ref.at[pl.ds(safe, 1), :],
                    stage.at[pl.ds(t, 1), :],
                    sem_g.at[c],
                ).start()
                return 0
            lax.fori_loop(0, S, issue_one, 0, unroll=8)

        def wait_chunk(c):
            def wait_one(j, _):
                pltpu.make_async_copy(
                    e_ref.at[pl.ds(0, 1), :],
                    stage.at[pl.ds(0, 1), :],
                    sem_g.at[c],
                ).wait()
                return 0
            lax.fori_loop(0, S, wait_one, 0, unroll=8)

        def xrdma(c):
            return pltpu.make_async_remote_copy(
                src_ref=xsend.at[pl.ds(c * S, S), :],
                dst_ref=xrecv.at[pl.ds(c * S, S), :],
                send_sem=sem_xs.at[c], recv_sem=sem_xr.at[c],
                device_id=(nx, my_y), device_id_type=pl.DeviceIdType.MESH,
            )

        def yrdma(c):
            return pltpu.make_async_remote_copy(
                src_ref=ysend.at[pl.ds(c * S, S), :],
                dst_ref=yrecv.at[pl.ds(c * S, S), :],
                send_sem=sem_ys.at[c], recv_sem=sem_yr.at[c],
                device_id=(my_x, ny), device_id_type=pl.DeviceIdType.MESH,
            )

        issue_chunk(0)
        issue_chunk(1)

        barrier_sem = pltpu.get_barrier_semaphore()
        pl.semaphore_signal(barrier_sem, inc=1, device_id=(nx, my_y),
                            device_id_type=pl.DeviceIdType.MESH)
        pl.semaphore_signal(barrier_sem, inc=1, device_id=(my_x, ny),
                            device_id_type=pl.DeviceIdType.MESH)
        pl.semaphore_wait(barrier_sem, 2)

        for c in range(C):
            if c + 2 < C:
                issue_chunk(c + 2)
            wait_chunk(c)
            sl = pl.ds(c * S, S)
            idv = idv_ref[pl.ds(base + c * S, S), :]
            mask = jnp.logical_and(idv >= voff, idv < voff + Vp)
            xsend[sl, :] = jnp.where(mask, stage[sl, :], 0.0).astype(
                jnp.bfloat16)
            xrdma(c).start()

        for c in range(C):
            xrdma(c).wait_recv()
            sl = pl.ds(c * S, S)
            red = xsend[sl, :] + xrecv[sl, :]
            ysend[sl, :] = red
            yrdma(c).start()
            obuf[pl.ds(c * S, S), :] = red.astype(jnp.float32)
            pltpu.make_async_copy(
                obuf.at[pl.ds(c * S, S), :],
                out_ref.at[pl.ds(base + c * S, S), :],
                sem_o,
            ).start()

        for c in range(C):
            yrdma(c).wait_recv()
            sl = pl.ds(c * S, S)
            obuf[pl.ds(Th + c * S, S), :] = yrecv[sl, :].astype(jnp.float32)
            pltpu.make_async_copy(
                obuf.at[pl.ds(Th + c * S, S), :],
                out_ref.at[pl.ds(ny * Th + c * S, S), :],
                sem_o,
            ).start()

        for c in range(C):
            xrdma(c).wait_send()
            yrdma(c).wait_send()
        for c in range(2 * C):
            pltpu.make_async_copy(
                obuf.at[pl.ds(0, S), :],
                out_ref.at[pl.ds(0, S), :],
                sem_o,
            ).wait()

    return pl.pallas_call(
        body,
        out_shape=jax.ShapeDtypeStruct((T, D), jnp.float32),
        in_specs=[
            pl.BlockSpec(memory_space=pltpu.SMEM),
            pl.BlockSpec(memory_space=pltpu.VMEM),
            pl.BlockSpec(memory_space=pl.ANY),
        ],
        out_specs=pl.BlockSpec(memory_space=pl.ANY),
        scratch_shapes=[
            pltpu.VMEM((Th, D), jnp.float32),
            pltpu.VMEM((Th, D), jnp.bfloat16),
            pltpu.VMEM((Th, D), jnp.bfloat16),
            pltpu.VMEM((Th, D), jnp.bfloat16),
            pltpu.VMEM((Th, D), jnp.bfloat16),
            pltpu.VMEM((T, D), jnp.float32),
            pltpu.SemaphoreType.DMA((C,)),
            pltpu.SemaphoreType.DMA,
            pltpu.SemaphoreType.DMA((C,)),
            pltpu.SemaphoreType.DMA((C,)),
            pltpu.SemaphoreType.DMA((C,)),
            pltpu.SemaphoreType.DMA((C,)),
        ],
        compiler_params=pltpu.CompilerParams(collective_id=0),
    )(ids, ids2d, E)


# device time: 7535 ns/iter; 6.4612x vs baseline; 4.1437x over previous
import os

import jax
import jax.numpy as jnp
from jax import lax
from jax.experimental import pallas as pl
from jax.experimental.pallas import tpu as pltpu

C = 8
_ABLATE = os.environ.get("ABLATE", "")


def kernel(ids, E):
    (T,) = ids.shape
    Vp, D = E.shape
    Th = T // 2
    S = Th // C

    ids2d = ids.reshape(T, 1)

    def body(ids_ref, idv_ref, e_ref, out_ref,
             stage, xsend, xrecv, ysend, yrecv, obuf, counts,
             sem_g, sem_o, sem_xs, sem_xr, sem_ys, sem_yr):
        my_x = lax.axis_index("x")
        my_y = lax.axis_index("y")
        nx = 1 - my_x
        ny = 1 - my_y

        base = my_y * Th
        voff = my_x * Vp

        def issue_chunk(c):
            if _ABLATE == "gather":
                return
            def issue_one(j, n):
                t = c * S + j
                idx = ids_ref[base + t]
                rel = idx - voff
                valid = jnp.logical_and(rel >= 0, rel < Vp)
                @pl.when(valid)
                def _():
                    pltpu.make_async_copy(
                        e_ref.at[pl.ds(rel, 1), :],
                        stage.at[pl.ds(t, 1), :],
                        sem_g.at[c],
                    ).start()
                return n + valid.astype(jnp.int32)
            n = lax.fori_loop(0, S, issue_one, jnp.int32(0), unroll=8)
            counts[c] = n

        def wait_chunk(c):
            if _ABLATE == "gather":
                return
            def wait_one(j, _):
                pltpu.make_async_copy(
                    e_ref.at[pl.ds(0, 1), :],
                    stage.at[pl.ds(0, 1), :],
                    sem_g.at[c],
                ).wait()
                return 0
            lax.fori_loop(0, counts[c], wait_one, 0)

        def xrdma(c):
            return pltpu.make_async_remote_copy(
                src_ref=xsend.at[pl.ds(c * S, S), :],
                dst_ref=xrecv.at[pl.ds(c * S, S), :],
                send_sem=sem_xs.at[c], recv_sem=sem_xr.at[c],
                device_id=(nx, my_y), device_id_type=pl.DeviceIdType.MESH,
            )

        def yrdma(c):
            return pltpu.make_async_remote_copy(
                src_ref=ysend.at[pl.ds(c * S, S), :],
                dst_ref=yrecv.at[pl.ds(c * S, S), :],
                send_sem=sem_ys.at[c], recv_sem=sem_yr.at[c],
                device_id=(my_x, ny), device_id_type=pl.DeviceIdType.MESH,
            )

        if _ABLATE in ("g1", "g2"):
            bsem = pltpu.get_barrier_semaphore()
            for nbr in [(nx, my_y), (my_x, ny)]:
                pl.semaphore_signal(bsem, inc=1, device_id=nbr,
                                    device_id_type=pl.DeviceIdType.MESH)
            pl.semaphore_wait(bsem, 2)
            if _ABLATE == "g1":
                for c in range(C):
                    issue_chunk(c)
                for c in range(C):
                    wait_chunk(c)
            for h in range(2):
                pltpu.make_async_copy(
                    stage.at[:, :],
                    out_ref.at[pl.ds(h * Th, Th), :],
                    sem_o,
                ).start()
            for h in range(2):
                pltpu.make_async_copy(
                    stage.at[:, :],
                    out_ref.at[pl.ds(0, Th), :],
                    sem_o,
                ).wait()
            return

        issue_chunk(0)
        issue_chunk(1)

        barrier_sem = pltpu.get_barrier_semaphore()
        pl.semaphore_signal(barrier_sem, inc=1, device_id=(nx, my_y),
                            device_id_type=pl.DeviceIdType.MESH)
        pl.semaphore_signal(barrier_sem, inc=1, device_id=(my_x, ny),
                            device_id_type=pl.DeviceIdType.MESH)
        pl.semaphore_wait(barrier_sem, 2)

        for c in range(C):
            wait_chunk(c)
            sl = pl.ds(c * S, S)
            idv = idv_ref[pl.ds(base + c * S, S), :]
            mask = jnp.logical_and(idv >= voff, idv < voff + Vp)
            xsend[sl, :] = jnp.where(mask, stage[sl, :], 0.0).astype(
                jnp.bfloat16)
            if _ABLATE != "comm":
                xrdma(c).start()
            if c + 2 < C:
                issue_chunk(c + 2)

        for c in range(C):
            sl = pl.ds(c * S, S)
            if _ABLATE != "comm":
                xrdma(c).wait_recv()
                red = xsend[sl, :] + xrecv[sl, :]
            else:
                red = xsend[sl, :] + xsend[sl, :]
            ysend[sl, :] = red
            if _ABLATE != "comm":
                yrdma(c).start()
            obuf[pl.ds(c * S, S), :] = red.astype(jnp.float32)
            pltpu.make_async_copy(
                obuf.at[pl.ds(c * S, S), :],
                out_ref.at[pl.ds(base + c * S, S), :],
                sem_o,
            ).start()

        for c in range(C):
            sl = pl.ds(c * S, S)
            if _ABLATE != "comm":
                yrdma(c).wait_recv()
                half2 = yrecv[sl, :]
            else:
                half2 = ysend[sl, :]
            obuf[pl.ds(Th + c * S, S), :] = half2.astype(jnp.float32)
            pltpu.make_async_copy(
                obuf.at[pl.ds(Th + c * S, S), :],
                out_ref.at[pl.ds(ny * Th + c * S, S), :],
                sem_o,
            ).start()

        if _ABLATE != "comm":
            for c in range(C):
                xrdma(c).wait_send()
                yrdma(c).wait_send()
        for c in range(2 * C):
            pltpu.make_async_copy(
                obuf.at[pl.ds(0, S), :],
                out_ref.at[pl.ds(0, S), :],
                sem_o,
            ).wait()

    return pl.pallas_call(
        body,
        out_shape=jax.ShapeDtypeStruct((T, D), jnp.float32),
        in_specs=[
            pl.BlockSpec(memory_space=pltpu.SMEM),
            pl.BlockSpec(memory_space=pltpu.VMEM),
            pl.BlockSpec(memory_space=pl.ANY),
        ],
        out_specs=pl.BlockSpec(memory_space=pl.ANY),
        scratch_shapes=[
            pltpu.VMEM((Th, D), jnp.float32),
            pltpu.VMEM((Th, D), jnp.bfloat16),
            pltpu.VMEM((Th, D), jnp.bfloat16),
            pltpu.VMEM((Th, D), jnp.bfloat16),
            pltpu.VMEM((Th, D), jnp.bfloat16),
            pltpu.VMEM((T, D), jnp.float32),
            pltpu.SMEM((C,), jnp.int32),
            pltpu.SemaphoreType.DMA((C,)),
            pltpu.SemaphoreType.DMA,
            pltpu.SemaphoreType.DMA((C,)),
            pltpu.SemaphoreType.DMA((C,)),
            pltpu.SemaphoreType.DMA((C,)),
            pltpu.SemaphoreType.DMA((C,)),
        ],
        compiler_params=pltpu.CompilerParams(collective_id=0),
    )(ids, ids2d, E)
